# baseline (speedup 1.0000x reference)
import functools

import jax
import jax.numpy as jnp
from jax import lax
from jax.experimental import pallas as pl
from jax.experimental.pallas import tpu as pltpu
from jax.experimental.pallas import tpu_sc as plsc

NC = 2
NS = 16
NW = NC * NS
K = 128
B = 1000

THETA = ((3.0, -3.0, 0.75), (0.0, 3.0, -1.5), (0.0, 0.0, 0.75))
DEG_W = 8


def _round8(v):
    return ((v + 7) // 8) * 8


def _seg_sum_sc(n_nodes, n_chunks, width, ring=True):
    cpw = n_chunks // NW
    nbuf = 4 if ring else 1
    ra = _round8(-(-n_nodes // NS))
    rb = n_nodes - ra * (NS - 1)
    acc_rows = n_nodes + 16
    mesh = plsc.VectorSubcoreMesh(
        core_axis_name="c", subcore_axis_name="s",
        num_cores=NC, num_subcores=NS)

    @functools.partial(
        pl.kernel,
        out_type=jax.ShapeDtypeStruct((NC * n_nodes, width), jnp.float32),
        mesh=mesh,
        compiler_params=pltpu.CompilerParams(use_tc_tiling_on_sc=False),
        scratch_types=[
            pltpu.VMEM((cpw, K), jnp.int32),
            pltpu.VMEM((cpw, K), jnp.int32),
            [pltpu.VMEM((K, width), jnp.float32) for _ in range(nbuf)],
            [pltpu.SemaphoreType.DMA for _ in range(nbuf)],
            [pltpu.SemaphoreType.DMA for _ in range(nbuf)],
            pltpu.VMEM_SHARED((acc_rows, width), jnp.float32),
        ],
    )
    def seg(table_hbm, src_hbm, dst_hbm, zeros_hbm, out_hbm,
            src_v, dst_v, rows, gsem, ssem, acc_sh):
        c = lax.axis_index("c")
        s = lax.axis_index("s")
        w = s * NC + c
        pltpu.sync_copy(src_hbm.at[pl.ds(w * cpw, cpw)], src_v)
        pltpu.sync_copy(dst_hbm.at[pl.ds(w * cpw, cpw)], dst_v)
        @pl.when(s < NS - 1)
        def _():
            pltpu.sync_copy(zeros_hbm.at[pl.ds(0, ra)],
                            acc_sh.at[pl.ds(s * ra, ra)])

        @pl.when(s == NS - 1)
        def _():
            pltpu.sync_copy(zeros_hbm.at[pl.ds(0, rb)],
                            acc_sh.at[pl.ds((NS - 1) * ra, rb)])

        @pl.when(s == 0)
        def _():
            pltpu.sync_copy(zeros_hbm.at[pl.ds(0, 16)],
                            acc_sh.at[pl.ds(n_nodes, 16)])

        plsc.subcore_barrier()

        if ring:
            pltpu.async_copy(table_hbm.at[src_v.at[0]], rows[0], gsem[0])
            pltpu.async_copy(table_hbm.at[src_v.at[1]], rows[1], gsem[1])

            def body(jj, carry):
                for b in range(4):
                    j = jj * 4 + b
                    bn = (b + 2) % 4
                    pltpu.make_async_copy(
                        table_hbm.at[pl.ds(0, K)], rows[b], gsem[b]).wait()
                    pltpu.async_copy(
                        rows[b], acc_sh.at[dst_v.at[j]], ssem[b], add=True)

                    @pl.when(j >= 2)
                    def _():
                        pltpu.make_async_copy(
                            rows[bn], acc_sh.at[pl.ds(0, K)],
                            ssem[bn]).wait()

                    @pl.when(j + 2 < cpw)
                    def _():
                        pltpu.async_copy(
                            table_hbm.at[src_v.at[j + 2]], rows[bn],
                            gsem[bn])
                return carry

            lax.fori_loop(0, cpw // 4, body, 0)
            pltpu.make_async_copy(
                rows[(cpw - 2) % 4], acc_sh.at[pl.ds(0, K)],
                ssem[(cpw - 2) % 4]).wait()
            pltpu.make_async_copy(
                rows[(cpw - 1) % 4], acc_sh.at[pl.ds(0, K)],
                ssem[(cpw - 1) % 4]).wait()
        else:
            def body(j, carry):
                pltpu.async_copy(
                    table_hbm.at[src_v.at[j]], rows[0], gsem[0]).wait()
                pltpu.sync_copy(rows[0], acc_sh.at[dst_v.at[j]], add=True)
                return carry

            lax.fori_loop(0, cpw, body, 0)
        plsc.subcore_barrier()

        @pl.when(s < NS - 1)
        def _():
            pltpu.sync_copy(acc_sh.at[pl.ds(s * ra, ra)],
                            out_hbm.at[pl.ds(c * n_nodes + s * ra, ra)])

        @pl.when(s == NS - 1)
        def _():
            pltpu.sync_copy(
                acc_sh.at[pl.ds((NS - 1) * ra, rb)],
                out_hbm.at[pl.ds(c * n_nodes + (NS - 1) * ra, rb)])

    return seg


def _deg_sc(n_nodes, n_chunks):
    cpw = n_chunks // NW
    ra = _round8(-(-n_nodes // NS))
    rb = n_nodes - ra * (NS - 1)
    acc_rows = n_nodes + 16
    mesh = plsc.VectorSubcoreMesh(
        core_axis_name="c", subcore_axis_name="s",
        num_cores=NC, num_subcores=NS)

    @functools.partial(
        pl.kernel,
        out_type=jax.ShapeDtypeStruct((NC * n_nodes, DEG_W), jnp.float32),
        mesh=mesh,
        compiler_params=pltpu.CompilerParams(use_tc_tiling_on_sc=False),
        scratch_types=[
            pltpu.VMEM((cpw, K), jnp.int32),
            pltpu.VMEM((K, DEG_W), jnp.float32),
            pltpu.VMEM_SHARED((acc_rows, DEG_W), jnp.float32),
        ],
    )
    def deg(dst_hbm, zeros_hbm, ones_hbm, out_hbm, dst_v, ones_v, acc_sh):
        c = lax.axis_index("c")
        s = lax.axis_index("s")
        w = s * NC + c
        pltpu.sync_copy(dst_hbm.at[pl.ds(w * cpw, cpw)], dst_v)
        pltpu.sync_copy(ones_hbm, ones_v)

        @pl.when(s < NS - 1)
        def _():
            pltpu.sync_copy(zeros_hbm.at[pl.ds(0, ra)],
                            acc_sh.at[pl.ds(s * ra, ra)])

        @pl.when(s == NS - 1)
        def _():
            pltpu.sync_copy(zeros_hbm.at[pl.ds(0, rb)],
                            acc_sh.at[pl.ds((NS - 1) * ra, rb)])

        @pl.when(s == 0)
        def _():
            pltpu.sync_copy(zeros_hbm.at[pl.ds(0, 16)],
                            acc_sh.at[pl.ds(n_nodes, 16)])

        plsc.subcore_barrier()

        def body(j, carry):
            pltpu.sync_copy(ones_v, acc_sh.at[dst_v.at[j]], add=True)
            return carry

        lax.fori_loop(0, cpw, body, 0)
        plsc.subcore_barrier()

        @pl.when(s < NS - 1)
        def _():
            pltpu.sync_copy(acc_sh.at[pl.ds(s * ra, ra)],
                            out_hbm.at[pl.ds(c * n_nodes + s * ra, ra)])

        @pl.when(s == NS - 1)
        def _():
            pltpu.sync_copy(
                acc_sh.at[pl.ds((NS - 1) * ra, rb)],
                out_hbm.at[pl.ds(c * n_nodes + (NS - 1) * ra, rb)])

    return deg


def _tc1(n, in_f, h):
    nblk = n // B

    def body(p0, p1, feat, w1, b1, x_ref, g0_ref, dinv_ref):
        deg = p0[:, 0:1] + p1[:, 0:1]
        dinv = lax.rsqrt(jnp.maximum(deg, 1.0))
        x = jnp.maximum(
            jnp.dot(feat[...], w1[...], preferred_element_type=jnp.float32)
            + b1[...], 0.0)
        x_ref[...] = x
        g0_ref[...] = x * dinv
        dinv_ref[...] = jnp.broadcast_to(dinv, (B, DEG_W))

    return pl.pallas_call(
        body,
        grid=(nblk,),
        in_specs=[
            pl.BlockSpec((B, DEG_W), lambda i: (i, 0)),
            pl.BlockSpec((B, DEG_W), lambda i, _n=nblk: (i + _n, 0)),
            pl.BlockSpec((B, in_f), lambda i: (i, 0)),
            pl.BlockSpec((in_f, h), lambda i: (0, 0)),
            pl.BlockSpec((1, h), lambda i: (0, 0)),
        ],
        out_specs=[
            pl.BlockSpec((B, h), lambda i: (i, 0)),
            pl.BlockSpec((B, h), lambda i: (i, 0)),
            pl.BlockSpec((B, DEG_W), lambda i: (i, 0)),
        ],
        out_shape=[
            jax.ShapeDtypeStruct((n, h), jnp.float32),
            jax.ShapeDtypeStruct((n, h), jnp.float32),
            jax.ShapeDtypeStruct((n, DEG_W), jnp.float32),
        ],
    )


def _tc_mid(n, d):
    nblk = n // B

    def body(p0, p1, f, dinv16, ad_ref, g_ref):
        dinv = dinv16[:, 0:1]
        ad = (p0[...] + p1[...]) * dinv
        ad_ref[...] = ad
        g_ref[...] = (f[...] - ad) * dinv

    return pl.pallas_call(
        body,
        grid=(nblk,),
        in_specs=[
            pl.BlockSpec((B, d), lambda i: (i, 0)),
            pl.BlockSpec((B, d), lambda i, _n=nblk: (i + _n, 0)),
            pl.BlockSpec((B, d), lambda i: (i, 0)),
            pl.BlockSpec((B, DEG_W), lambda i: (i, 0)),
        ],
        out_specs=[
            pl.BlockSpec((B, d), lambda i: (i, 0)),
            pl.BlockSpec((B, d), lambda i: (i, 0)),
        ],
        out_shape=[
            jax.ShapeDtypeStruct((n, d), jnp.float32),
            jax.ShapeDtypeStruct((n, d), jnp.float32),
        ],
    )


def _tc3(n, h, z, in_f):
    nblk = n // B
    assert in_f == 2 * h

    def body(p0, p1, x, a1d, dinv16, eps, vsum, v12, v2, b2, repw, repb,
             wf, bf, y_ref, gy_ref, kl_ref):
        i = pl.program_id(0)
        dinv = dinv16[:, 0:1]
        a2d = (p0[...] + p1[...]) * dinv
        dot = functools.partial(jnp.dot, preferred_element_type=jnp.float32)
        pos = (dot(x[...], vsum[...]) - dot(a1d[...], v12[...])
               - dot(a2d, v2[...]) + b2[...])
        mu = dot(pos, repw[...]) + repb[...]
        zz = mu + eps[...] * jnp.exp(mu * 0.5)
        y = jnp.maximum(dot(zz, wf[...]) + bf[...], 0.0)
        y_ref[...] = y
        gy_ref[...] = y * dinv

        @pl.when(i == 0)
        def _():
            kl_ref[...] = jnp.zeros((1, 1), jnp.float32)

        kl_ref[...] += jnp.sum(1.0 + mu - mu * mu - jnp.exp(mu)).reshape(1, 1)

    return pl.pallas_call(
        body,
        grid=(nblk,),
        in_specs=[
            pl.BlockSpec((B, h), lambda i: (i, 0)),
            pl.BlockSpec((B, h), lambda i, _n=nblk: (i + _n, 0)),
            pl.BlockSpec((B, h), lambda i: (i, 0)),
            pl.BlockSpec((B, h), lambda i: (i, 0)),
            pl.BlockSpec((B, DEG_W), lambda i: (i, 0)),
            pl.BlockSpec((B, z), lambda i: (i, 0)),
            pl.BlockSpec((h, h), lambda i: (0, 0)),
            pl.BlockSpec((h, h), lambda i: (0, 0)),
            pl.BlockSpec((h, h), lambda i: (0, 0)),
            pl.BlockSpec((1, h), lambda i: (0, 0)),
            pl.BlockSpec((h, z), lambda i: (0, 0)),
            pl.BlockSpec((1, z), lambda i: (0, 0)),
            pl.BlockSpec((z, in_f), lambda i: (0, 0)),
            pl.BlockSpec((1, in_f), lambda i: (0, 0)),
        ],
        out_specs=[
            pl.BlockSpec((B, in_f), lambda i: (i, 0)),
            pl.BlockSpec((B, in_f), lambda i: (i, 0)),
            pl.BlockSpec((1, 1), lambda i: (0, 0)),
        ],
        out_shape=[
            jax.ShapeDtypeStruct((n, in_f), jnp.float32),
            jax.ShapeDtypeStruct((n, in_f), jnp.float32),
            jax.ShapeDtypeStruct((1, 1), jnp.float32),
        ],
    )


def _tc5(n, in_f):
    nblk = n // B

    def body(p0, p1, y, b1d, dinv16, feat, usum, u12, u2, db2, kl, out_ref):
        i = pl.program_id(0)
        dinv = dinv16[:, 0:1]
        b2d = (p0[...] + p1[...]) * dinv
        dot = functools.partial(jnp.dot, preferred_element_type=jnp.float32)
        xr = (dot(y[...], usum[...]) - dot(b1d[...], u12[...])
              - dot(b2d, u2[...]) + db2[...])
        d = xr - feat[...]

        @pl.when(i == 0)
        def _():
            out_ref[...] = -0.5 * kl[...]

        out_ref[...] += jnp.sum(d * d).reshape(1, 1)

    return pl.pallas_call(
        body,
        grid=(nblk,),
        in_specs=[
            pl.BlockSpec((B, in_f), lambda i: (i, 0)),
            pl.BlockSpec((B, in_f), lambda i, _n=nblk: (i + _n, 0)),
            pl.BlockSpec((B, in_f), lambda i: (i, 0)),
            pl.BlockSpec((B, in_f), lambda i: (i, 0)),
            pl.BlockSpec((B, DEG_W), lambda i: (i, 0)),
            pl.BlockSpec((B, in_f), lambda i: (i, 0)),
            pl.BlockSpec((in_f, in_f), lambda i: (0, 0)),
            pl.BlockSpec((in_f, in_f), lambda i: (0, 0)),
            pl.BlockSpec((in_f, in_f), lambda i: (0, 0)),
            pl.BlockSpec((1, in_f), lambda i: (0, 0)),
            pl.BlockSpec((1, 1), lambda i: (0, 0)),
        ],
        out_specs=pl.BlockSpec((1, 1), lambda i: (0, 0)),
        out_shape=jax.ShapeDtypeStruct((1, 1), jnp.float32),
    )


def _fold_theta(w2, d):
    w2r = w2.reshape(3, d, w2.shape[1])

    def mix(coefs):
        return (coefs[0] * w2r[0] + coefs[1] * w2r[1] + coefs[2] * w2r[2])

    vs = [mix([THETA[i][k] for i in range(3)]) for k in range(3)]
    return vs[0] + vs[1] + vs[2], vs[1] + vs[2], vs[2]


def kernel(features, edge_index, enc_W1, enc_b1, enc_W2, enc_b2, rep_W,
           rep_b, rec_W, rec_b, dec_W1, dec_b1, dec_W2, dec_b2, disc_W):
    n, in_f = features.shape
    h = enc_W1.shape[1]
    z = rep_W.shape[1]
    e = edge_index.shape[1]

    n_chunks = -(-e // K)
    n_chunks = ((n_chunks + NW * 8 - 1) // (NW * 8)) * (NW * 8)
    pad = n_chunks * K - e
    src = jnp.concatenate(
        [edge_index[0].astype(jnp.int32), jnp.zeros((pad,), jnp.int32)]
    ).reshape(n_chunks, K)
    dst = jnp.concatenate(
        [edge_index[1].astype(jnp.int32), jnp.full((pad,), n, jnp.int32)]
    ).reshape(n_chunks, K)

    ra = _round8(-(-n // NS))
    zeros_h = jnp.zeros((ra, h), jnp.float32)
    zeros_f = jnp.zeros((ra, in_f), jnp.float32)
    zeros_d = jnp.zeros((ra, DEG_W), jnp.float32)
    ones_d = jnp.ones((K, DEG_W), jnp.float32)

    vsum, v12, v2 = _fold_theta(enc_W2, h)
    usum, u12, u2 = _fold_theta(dec_W2, in_f)
    wf = rec_W @ dec_W1
    bf = (rec_b @ dec_W1 + dec_b1).reshape(1, in_f)
    b1r = enc_b1.reshape(1, h)
    b2r = enc_b2.reshape(1, h)
    db2r = dec_b2.reshape(1, in_f)
    repbr = rep_b.reshape(1, z)

    keps = jax.random.split(jax.random.key(42))[1]
    eps = jax.random.normal(keps, (n, z), dtype=jnp.float32)

    seg_h = _seg_sum_sc(n, n_chunks, h, ring=True)
    seg_f = _seg_sum_sc(n, n_chunks, in_f, ring=False)

    deg_p = _deg_sc(n, n_chunks)(dst, zeros_d, ones_d)
    x, g0, dinv16 = _tc1(n, in_f, h)(deg_p, deg_p, features, enc_W1, b1r)
    a1_p = seg_h(g0, src, dst, zeros_h)
    a1d, g1 = _tc_mid(n, h)(a1_p, a1_p, x, dinv16)
    a2_p = seg_h(g1, src, dst, zeros_h)
    y, gy0, kl = _tc3(n, h, z, in_f)(
        a2_p, a2_p, x, a1d, dinv16, eps, vsum, v12, v2, b2r, rep_W, repbr,
        wf, bf)
    b1_p = seg_f(gy0, src, dst, zeros_f)
    b1d, gy1 = _tc_mid(n, in_f)(b1_p, b1_p, y, dinv16)
    b2_p = seg_f(gy1, src, dst, zeros_f)
    out = _tc5(n, in_f)(b2_p, b2_p, y, b1d, dinv16, features, usum, u12,
                        u2, db2r, kl)
    return out[0, 0]


# speedup vs baseline: 1.0993x; 1.0993x over previous
import functools

import jax
import jax.numpy as jnp
from jax import lax
from jax.experimental import pallas as pl
from jax.experimental.pallas import tpu as pltpu
from jax.experimental.pallas import tpu_sc as plsc

NC = 2
NS = 16
NW = NC * NS
K = 128
B = 1000

THETA = ((3.0, -3.0, 0.75), (0.0, 3.0, -1.5), (0.0, 0.0, 0.75))
DEG_W = 16


def _round8(v):
    return ((v + 7) // 8) * 8


def _seg_sum_sc(n_nodes, n_chunks, width, ring=True):
    cpw = n_chunks // NW
    nbuf = 4 if ring else 1
    ra = _round8(-(-n_nodes // NS))
    rb = n_nodes - ra * (NS - 1)
    acc_rows = n_nodes + 16
    mesh = plsc.VectorSubcoreMesh(
        core_axis_name="c", subcore_axis_name="s",
        num_cores=NC, num_subcores=NS)

    @functools.partial(
        pl.kernel,
        out_type=jax.ShapeDtypeStruct((NC * n_nodes, width), jnp.float32),
        mesh=mesh,
        compiler_params=pltpu.CompilerParams(use_tc_tiling_on_sc=False),
        scratch_types=[
            pltpu.VMEM((cpw, K), jnp.int32),
            pltpu.VMEM((cpw, K), jnp.int32),
            [pltpu.VMEM((K, width), jnp.float32) for _ in range(nbuf)],
            [pltpu.SemaphoreType.DMA for _ in range(nbuf)],
            [pltpu.SemaphoreType.DMA for _ in range(nbuf)],
            pltpu.VMEM_SHARED((acc_rows, width), jnp.float32),
        ],
    )
    def seg(table_hbm, src_hbm, dst_hbm, zeros_hbm, out_hbm,
            src_v, dst_v, rows, gsem, ssem, acc_sh):
        c = lax.axis_index("c")
        s = lax.axis_index("s")
        w = s * NC + c
        pltpu.sync_copy(src_hbm.at[pl.ds(w * cpw, cpw)], src_v)
        pltpu.sync_copy(dst_hbm.at[pl.ds(w * cpw, cpw)], dst_v)
        @pl.when(s < NS - 1)
        def _():
            pltpu.sync_copy(zeros_hbm.at[pl.ds(0, ra)],
                            acc_sh.at[pl.ds(s * ra, ra)])

        @pl.when(s == NS - 1)
        def _():
            pltpu.sync_copy(zeros_hbm.at[pl.ds(0, rb)],
                            acc_sh.at[pl.ds((NS - 1) * ra, rb)])

        @pl.when(s == 0)
        def _():
            pltpu.sync_copy(zeros_hbm.at[pl.ds(0, 16)],
                            acc_sh.at[pl.ds(n_nodes, 16)])

        plsc.subcore_barrier()

        if ring:
            pltpu.async_copy(table_hbm.at[src_v.at[0]], rows[0], gsem[0])
            pltpu.async_copy(table_hbm.at[src_v.at[1]], rows[1], gsem[1])

            def body(jj, carry):
                for b in range(4):
                    j = jj * 4 + b
                    bn = (b + 2) % 4
                    pltpu.make_async_copy(
                        table_hbm.at[pl.ds(0, K)], rows[b], gsem[b]).wait()
                    pltpu.async_copy(
                        rows[b], acc_sh.at[dst_v.at[j]], ssem[b], add=True)

                    @pl.when(j >= 2)
                    def _():
                        pltpu.make_async_copy(
                            rows[bn], acc_sh.at[pl.ds(0, K)],
                            ssem[bn]).wait()

                    @pl.when(j + 2 < cpw)
                    def _():
                        pltpu.async_copy(
                            table_hbm.at[src_v.at[j + 2]], rows[bn],
                            gsem[bn])
                return carry

            lax.fori_loop(0, cpw // 4, body, 0)
            pltpu.make_async_copy(
                rows[(cpw - 2) % 4], acc_sh.at[pl.ds(0, K)],
                ssem[(cpw - 2) % 4]).wait()
            pltpu.make_async_copy(
                rows[(cpw - 1) % 4], acc_sh.at[pl.ds(0, K)],
                ssem[(cpw - 1) % 4]).wait()
        else:
            def body(j, carry):
                pltpu.async_copy(
                    table_hbm.at[src_v.at[j]], rows[0], gsem[0]).wait()
                pltpu.sync_copy(rows[0], acc_sh.at[dst_v.at[j]], add=True)
                return carry

            lax.fori_loop(0, cpw, body, 0)
        plsc.subcore_barrier()

        @pl.when(s < NS - 1)
        def _():
            pltpu.sync_copy(acc_sh.at[pl.ds(s * ra, ra)],
                            out_hbm.at[pl.ds(c * n_nodes + s * ra, ra)])

        @pl.when(s == NS - 1)
        def _():
            pltpu.sync_copy(
                acc_sh.at[pl.ds((NS - 1) * ra, rb)],
                out_hbm.at[pl.ds(c * n_nodes + (NS - 1) * ra, rb)])

    return seg


def _deg_sc(n_nodes, n_chunks):
    cpw = n_chunks // NW
    ra = _round8(-(-n_nodes // NS))
    rb = n_nodes - ra * (NS - 1)
    acc_rows = n_nodes + 16
    mesh = plsc.VectorSubcoreMesh(
        core_axis_name="c", subcore_axis_name="s",
        num_cores=NC, num_subcores=NS)

    @functools.partial(
        pl.kernel,
        out_type=jax.ShapeDtypeStruct((NC * n_nodes, DEG_W), jnp.float32),
        mesh=mesh,
        compiler_params=pltpu.CompilerParams(use_tc_tiling_on_sc=False),
        scratch_types=[
            pltpu.VMEM((cpw, K), jnp.int32),
            pltpu.VMEM((K, DEG_W), jnp.float32),
            pltpu.VMEM_SHARED((acc_rows, DEG_W), jnp.float32),
        ],
    )
    def deg(dst_hbm, zeros_hbm, ones_hbm, out_hbm, dst_v, ones_v, acc_sh):
        c = lax.axis_index("c")
        s = lax.axis_index("s")
        w = s * NC + c
        pltpu.sync_copy(dst_hbm.at[pl.ds(w * cpw, cpw)], dst_v)
        pltpu.sync_copy(ones_hbm, ones_v)

        @pl.when(s < NS - 1)
        def _():
            pltpu.sync_copy(zeros_hbm.at[pl.ds(0, ra)],
                            acc_sh.at[pl.ds(s * ra, ra)])

        @pl.when(s == NS - 1)
        def _():
            pltpu.sync_copy(zeros_hbm.at[pl.ds(0, rb)],
                            acc_sh.at[pl.ds((NS - 1) * ra, rb)])

        @pl.when(s == 0)
        def _():
            pltpu.sync_copy(zeros_hbm.at[pl.ds(0, 16)],
                            acc_sh.at[pl.ds(n_nodes, 16)])

        plsc.subcore_barrier()

        def body(j, carry):
            pltpu.sync_copy(ones_v, acc_sh.at[dst_v.at[j]], add=True)
            return carry

        lax.fori_loop(0, cpw, body, 0)
        plsc.subcore_barrier()

        @pl.when(s < NS - 1)
        def _():
            pltpu.sync_copy(acc_sh.at[pl.ds(s * ra, ra)],
                            out_hbm.at[pl.ds(c * n_nodes + s * ra, ra)])

        @pl.when(s == NS - 1)
        def _():
            pltpu.sync_copy(
                acc_sh.at[pl.ds((NS - 1) * ra, rb)],
                out_hbm.at[pl.ds(c * n_nodes + (NS - 1) * ra, rb)])

    return deg


def _tc1(n, in_f, h):
    nblk = n // B

    def body(p0, p1, feat, w1, b1, x_ref, g0_ref, dinv_ref):
        deg = p0[:, 0:1] + p1[:, 0:1]
        dinv = lax.rsqrt(jnp.maximum(deg, 1.0))
        x = jnp.maximum(
            jnp.dot(feat[...], w1[...], preferred_element_type=jnp.float32)
            + b1[...], 0.0)
        x_ref[...] = x
        g0_ref[...] = x * dinv
        dinv_ref[...] = jnp.broadcast_to(dinv, (B, DEG_W))

    return pl.pallas_call(
        body,
        grid=(nblk,),
        in_specs=[
            pl.BlockSpec((B, DEG_W), lambda i: (i, 0)),
            pl.BlockSpec((B, DEG_W), lambda i, _n=nblk: (i + _n, 0)),
            pl.BlockSpec((B, in_f), lambda i: (i, 0)),
            pl.BlockSpec((in_f, h), lambda i: (0, 0)),
            pl.BlockSpec((1, h), lambda i: (0, 0)),
        ],
        out_specs=[
            pl.BlockSpec((B, h), lambda i: (i, 0)),
            pl.BlockSpec((B, h), lambda i: (i, 0)),
            pl.BlockSpec((B, DEG_W), lambda i: (i, 0)),
        ],
        out_shape=[
            jax.ShapeDtypeStruct((n, h), jnp.float32),
            jax.ShapeDtypeStruct((n, h), jnp.float32),
            jax.ShapeDtypeStruct((n, DEG_W), jnp.float32),
        ],
    )


def _tc_mid(n, d):
    nblk = n // B

    def body(p0, p1, f, dinv16, ad_ref, g_ref):
        dinv = dinv16[:, 0:1]
        ad = (p0[...] + p1[...]) * dinv
        ad_ref[...] = ad
        g_ref[...] = (f[...] - ad) * dinv

    return pl.pallas_call(
        body,
        grid=(nblk,),
        in_specs=[
            pl.BlockSpec((B, d), lambda i: (i, 0)),
            pl.BlockSpec((B, d), lambda i, _n=nblk: (i + _n, 0)),
            pl.BlockSpec((B, d), lambda i: (i, 0)),
            pl.BlockSpec((B, DEG_W), lambda i: (i, 0)),
        ],
        out_specs=[
            pl.BlockSpec((B, d), lambda i: (i, 0)),
            pl.BlockSpec((B, d), lambda i: (i, 0)),
        ],
        out_shape=[
            jax.ShapeDtypeStruct((n, d), jnp.float32),
            jax.ShapeDtypeStruct((n, d), jnp.float32),
        ],
    )


def _tc3(n, h, z, in_f):
    nblk = n // B
    assert in_f == 2 * h

    def body(p0, p1, x, a1d, dinv16, eps, vsum, v12, v2, b2, repw, repb,
             wf, bf, y_ref, gy_ref, kl_ref):
        i = pl.program_id(0)
        dinv = dinv16[:, 0:1]
        a2d = (p0[...] + p1[...]) * dinv
        dot = functools.partial(jnp.dot, preferred_element_type=jnp.float32)
        pos = (dot(x[...], vsum[...]) - dot(a1d[...], v12[...])
               - dot(a2d, v2[...]) + b2[...])
        mu = dot(pos, repw[...]) + repb[...]
        zz = mu + eps[...] * jnp.exp(mu * 0.5)
        y = jnp.maximum(dot(zz, wf[...]) + bf[...], 0.0)
        y_ref[...] = y
        gy_ref[...] = y * dinv

        @pl.when(i == 0)
        def _():
            kl_ref[...] = jnp.zeros((1, 1), jnp.float32)

        kl_ref[...] += jnp.sum(1.0 + mu - mu * mu - jnp.exp(mu)).reshape(1, 1)

    return pl.pallas_call(
        body,
        grid=(nblk,),
        in_specs=[
            pl.BlockSpec((B, h), lambda i: (i, 0)),
            pl.BlockSpec((B, h), lambda i, _n=nblk: (i + _n, 0)),
            pl.BlockSpec((B, h), lambda i: (i, 0)),
            pl.BlockSpec((B, h), lambda i: (i, 0)),
            pl.BlockSpec((B, DEG_W), lambda i: (i, 0)),
            pl.BlockSpec((B, z), lambda i: (i, 0)),
            pl.BlockSpec((h, h), lambda i: (0, 0)),
            pl.BlockSpec((h, h), lambda i: (0, 0)),
            pl.BlockSpec((h, h), lambda i: (0, 0)),
            pl.BlockSpec((1, h), lambda i: (0, 0)),
            pl.BlockSpec((h, z), lambda i: (0, 0)),
            pl.BlockSpec((1, z), lambda i: (0, 0)),
            pl.BlockSpec((z, in_f), lambda i: (0, 0)),
            pl.BlockSpec((1, in_f), lambda i: (0, 0)),
        ],
        out_specs=[
            pl.BlockSpec((B, in_f), lambda i: (i, 0)),
            pl.BlockSpec((B, in_f), lambda i: (i, 0)),
            pl.BlockSpec((1, 1), lambda i: (0, 0)),
        ],
        out_shape=[
            jax.ShapeDtypeStruct((n, in_f), jnp.float32),
            jax.ShapeDtypeStruct((n, in_f), jnp.float32),
            jax.ShapeDtypeStruct((1, 1), jnp.float32),
        ],
    )


def _tc5(n, in_f):
    nblk = n // B

    def body(p0, p1, y, b1d, dinv16, feat, usum, u12, u2, db2, kl, out_ref):
        i = pl.program_id(0)
        dinv = dinv16[:, 0:1]
        b2d = (p0[...] + p1[...]) * dinv
        dot = functools.partial(jnp.dot, preferred_element_type=jnp.float32)
        xr = (dot(y[...], usum[...]) - dot(b1d[...], u12[...])
              - dot(b2d, u2[...]) + db2[...])
        d = xr - feat[...]

        @pl.when(i == 0)
        def _():
            out_ref[...] = -0.5 * kl[...]

        out_ref[...] += jnp.sum(d * d).reshape(1, 1)

    return pl.pallas_call(
        body,
        grid=(nblk,),
        in_specs=[
            pl.BlockSpec((B, in_f), lambda i: (i, 0)),
            pl.BlockSpec((B, in_f), lambda i, _n=nblk: (i + _n, 0)),
            pl.BlockSpec((B, in_f), lambda i: (i, 0)),
            pl.BlockSpec((B, in_f), lambda i: (i, 0)),
            pl.BlockSpec((B, DEG_W), lambda i: (i, 0)),
            pl.BlockSpec((B, in_f), lambda i: (i, 0)),
            pl.BlockSpec((in_f, in_f), lambda i: (0, 0)),
            pl.BlockSpec((in_f, in_f), lambda i: (0, 0)),
            pl.BlockSpec((in_f, in_f), lambda i: (0, 0)),
            pl.BlockSpec((1, in_f), lambda i: (0, 0)),
            pl.BlockSpec((1, 1), lambda i: (0, 0)),
        ],
        out_specs=pl.BlockSpec((1, 1), lambda i: (0, 0)),
        out_shape=jax.ShapeDtypeStruct((1, 1), jnp.float32),
    )


def _fold_theta(w2, d):
    w2r = w2.reshape(3, d, w2.shape[1])

    def mix(coefs):
        return (coefs[0] * w2r[0] + coefs[1] * w2r[1] + coefs[2] * w2r[2])

    vs = [mix([THETA[i][k] for i in range(3)]) for k in range(3)]
    return vs[0] + vs[1] + vs[2], vs[1] + vs[2], vs[2]


def kernel(features, edge_index, enc_W1, enc_b1, enc_W2, enc_b2, rep_W,
           rep_b, rec_W, rec_b, dec_W1, dec_b1, dec_W2, dec_b2, disc_W):
    n, in_f = features.shape
    h = enc_W1.shape[1]
    z = rep_W.shape[1]
    e = edge_index.shape[1]

    n_chunks = -(-e // K)
    n_chunks = ((n_chunks + NW * 8 - 1) // (NW * 8)) * (NW * 8)
    pad = n_chunks * K - e
    src = jnp.concatenate(
        [edge_index[0].astype(jnp.int32), jnp.zeros((pad,), jnp.int32)]
    ).reshape(n_chunks, K)
    dst = jnp.concatenate(
        [edge_index[1].astype(jnp.int32), jnp.full((pad,), n, jnp.int32)]
    ).reshape(n_chunks, K)

    ra = _round8(-(-n // NS))
    zeros_h = jnp.zeros((ra, h), jnp.float32)
    zeros_f = jnp.zeros((ra, in_f), jnp.float32)
    zeros_d = jnp.zeros((ra, DEG_W), jnp.float32)
    ones_d = jnp.ones((K, DEG_W), jnp.float32)

    vsum, v12, v2 = _fold_theta(enc_W2, h)
    usum, u12, u2 = _fold_theta(dec_W2, in_f)
    wf = rec_W @ dec_W1
    bf = (rec_b @ dec_W1 + dec_b1).reshape(1, in_f)
    b1r = enc_b1.reshape(1, h)
    b2r = enc_b2.reshape(1, h)
    db2r = dec_b2.reshape(1, in_f)
    repbr = rep_b.reshape(1, z)

    keps = jax.random.split(jax.random.key(42))[1]
    eps = jax.random.normal(keps, (n, z), dtype=jnp.float32)

    seg_h = _seg_sum_sc(n, n_chunks, h, ring=False)
    seg_f = _seg_sum_sc(n, n_chunks, in_f, ring=False)

    deg_p = _deg_sc(n, n_chunks)(dst, zeros_d, ones_d)
    x, g0, dinv16 = _tc1(n, in_f, h)(deg_p, deg_p, features, enc_W1, b1r)
    a1_p = seg_h(g0, src, dst, zeros_h)
    a1d, g1 = _tc_mid(n, h)(a1_p, a1_p, x, dinv16)
    a2_p = seg_h(g1, src, dst, zeros_h)
    y, gy0, kl = _tc3(n, h, z, in_f)(
        a2_p, a2_p, x, a1d, dinv16, eps, vsum, v12, v2, b2r, rep_W, repbr,
        wf, bf)
    b1_p = seg_f(gy0, src, dst, zeros_f)
    b1d, gy1 = _tc_mid(n, in_f)(b1_p, b1_p, y, dinv16)
    b2_p = seg_f(gy1, src, dst, zeros_f)
    out = _tc5(n, in_f)(b2_p, b2_p, y, b1d, dinv16, features, usum, u12,
                        u2, db2r, kl)
    return out[0, 0]


# speedup vs baseline: 1.1090x; 1.0088x over previous
import functools

import jax
import jax.numpy as jnp
from jax import lax
from jax.experimental import pallas as pl
from jax.experimental.pallas import tpu as pltpu
from jax.experimental.pallas import tpu_sc as plsc

NC = 2
NS = 16
NW = NC * NS
K = 128
B = 2000

THETA = ((3.0, -3.0, 0.75), (0.0, 3.0, -1.5), (0.0, 0.0, 0.75))
DEG_W = 16


def _round8(v):
    return ((v + 7) // 8) * 8


def _seg_sum_sc(n_nodes, n_chunks, width, ring=True):
    cpw = n_chunks // NW
    nbuf = 4 if ring else 1
    ra = _round8(-(-n_nodes // NS))
    rb = n_nodes - ra * (NS - 1)
    acc_rows = n_nodes + 16
    mesh = plsc.VectorSubcoreMesh(
        core_axis_name="c", subcore_axis_name="s",
        num_cores=NC, num_subcores=NS)

    @functools.partial(
        pl.kernel,
        out_type=jax.ShapeDtypeStruct((NC * n_nodes, width), jnp.float32),
        mesh=mesh,
        compiler_params=pltpu.CompilerParams(use_tc_tiling_on_sc=False),
        scratch_types=[
            pltpu.VMEM((cpw, K), jnp.int32),
            pltpu.VMEM((cpw, K), jnp.int32),
            [pltpu.VMEM((K, width), jnp.float32) for _ in range(nbuf)],
            [pltpu.SemaphoreType.DMA for _ in range(nbuf)],
            [pltpu.SemaphoreType.DMA for _ in range(nbuf)],
            pltpu.VMEM_SHARED((acc_rows, width), jnp.float32),
        ],
    )
    def seg(table_hbm, src_hbm, dst_hbm, zeros_hbm, out_hbm,
            src_v, dst_v, rows, gsem, ssem, acc_sh):
        c = lax.axis_index("c")
        s = lax.axis_index("s")
        w = s * NC + c
        pltpu.sync_copy(src_hbm.at[pl.ds(w * cpw, cpw)], src_v)
        pltpu.sync_copy(dst_hbm.at[pl.ds(w * cpw, cpw)], dst_v)
        @pl.when(s < NS - 1)
        def _():
            pltpu.sync_copy(zeros_hbm.at[pl.ds(0, ra)],
                            acc_sh.at[pl.ds(s * ra, ra)])

        @pl.when(s == NS - 1)
        def _():
            pltpu.sync_copy(zeros_hbm.at[pl.ds(0, rb)],
                            acc_sh.at[pl.ds((NS - 1) * ra, rb)])

        @pl.when(s == 0)
        def _():
            pltpu.sync_copy(zeros_hbm.at[pl.ds(0, 16)],
                            acc_sh.at[pl.ds(n_nodes, 16)])

        plsc.subcore_barrier()

        if ring:
            pltpu.async_copy(table_hbm.at[src_v.at[0]], rows[0], gsem[0])
            pltpu.async_copy(table_hbm.at[src_v.at[1]], rows[1], gsem[1])

            def body(jj, carry):
                for b in range(4):
                    j = jj * 4 + b
                    bn = (b + 2) % 4
                    pltpu.make_async_copy(
                        table_hbm.at[pl.ds(0, K)], rows[b], gsem[b]).wait()
                    pltpu.async_copy(
                        rows[b], acc_sh.at[dst_v.at[j]], ssem[b], add=True)

                    @pl.when(j >= 2)
                    def _():
                        pltpu.make_async_copy(
                            rows[bn], acc_sh.at[pl.ds(0, K)],
                            ssem[bn]).wait()

                    @pl.when(j + 2 < cpw)
                    def _():
                        pltpu.async_copy(
                            table_hbm.at[src_v.at[j + 2]], rows[bn],
                            gsem[bn])
                return carry

            lax.fori_loop(0, cpw // 4, body, 0)
            pltpu.make_async_copy(
                rows[(cpw - 2) % 4], acc_sh.at[pl.ds(0, K)],
                ssem[(cpw - 2) % 4]).wait()
            pltpu.make_async_copy(
                rows[(cpw - 1) % 4], acc_sh.at[pl.ds(0, K)],
                ssem[(cpw - 1) % 4]).wait()
        else:
            def body(j, carry):
                pltpu.async_copy(
                    table_hbm.at[src_v.at[j]], rows[0], gsem[0]).wait()
                pltpu.sync_copy(rows[0], acc_sh.at[dst_v.at[j]], add=True)
                return carry

            lax.fori_loop(0, cpw, body, 0)
        plsc.subcore_barrier()

        @pl.when(s < NS - 1)
        def _():
            pltpu.sync_copy(acc_sh.at[pl.ds(s * ra, ra)],
                            out_hbm.at[pl.ds(c * n_nodes + s * ra, ra)])

        @pl.when(s == NS - 1)
        def _():
            pltpu.sync_copy(
                acc_sh.at[pl.ds((NS - 1) * ra, rb)],
                out_hbm.at[pl.ds(c * n_nodes + (NS - 1) * ra, rb)])

    return seg


def _deg_sc(n_nodes, n_chunks):
    cpw = n_chunks // NW
    ra = _round8(-(-n_nodes // NS))
    rb = n_nodes - ra * (NS - 1)
    acc_rows = n_nodes + 16
    mesh = plsc.VectorSubcoreMesh(
        core_axis_name="c", subcore_axis_name="s",
        num_cores=NC, num_subcores=NS)

    @functools.partial(
        pl.kernel,
        out_type=jax.ShapeDtypeStruct((NC * n_nodes, DEG_W), jnp.float32),
        mesh=mesh,
        compiler_params=pltpu.CompilerParams(use_tc_tiling_on_sc=False),
        scratch_types=[
            pltpu.VMEM((cpw, K), jnp.int32),
            pltpu.VMEM((K, DEG_W), jnp.float32),
            pltpu.VMEM_SHARED((acc_rows, DEG_W), jnp.float32),
        ],
    )
    def deg(dst_hbm, zeros_hbm, ones_hbm, out_hbm, dst_v, ones_v, acc_sh):
        c = lax.axis_index("c")
        s = lax.axis_index("s")
        w = s * NC + c
        pltpu.sync_copy(dst_hbm.at[pl.ds(w * cpw, cpw)], dst_v)
        pltpu.sync_copy(ones_hbm, ones_v)

        @pl.when(s < NS - 1)
        def _():
            pltpu.sync_copy(zeros_hbm.at[pl.ds(0, ra)],
                            acc_sh.at[pl.ds(s * ra, ra)])

        @pl.when(s == NS - 1)
        def _():
            pltpu.sync_copy(zeros_hbm.at[pl.ds(0, rb)],
                            acc_sh.at[pl.ds((NS - 1) * ra, rb)])

        @pl.when(s == 0)
        def _():
            pltpu.sync_copy(zeros_hbm.at[pl.ds(0, 16)],
                            acc_sh.at[pl.ds(n_nodes, 16)])

        plsc.subcore_barrier()

        def body(j, carry):
            pltpu.sync_copy(ones_v, acc_sh.at[dst_v.at[j]], add=True)
            return carry

        lax.fori_loop(0, cpw, body, 0)
        plsc.subcore_barrier()

        @pl.when(s < NS - 1)
        def _():
            pltpu.sync_copy(acc_sh.at[pl.ds(s * ra, ra)],
                            out_hbm.at[pl.ds(c * n_nodes + s * ra, ra)])

        @pl.when(s == NS - 1)
        def _():
            pltpu.sync_copy(
                acc_sh.at[pl.ds((NS - 1) * ra, rb)],
                out_hbm.at[pl.ds(c * n_nodes + (NS - 1) * ra, rb)])

    return deg


def _tc1(n, in_f, h):
    nblk = n // B

    def body(p0, p1, feat, w1, b1, x_ref, g0_ref, dinv_ref):
        deg = p0[:, 0:1] + p1[:, 0:1]
        dinv = lax.rsqrt(jnp.maximum(deg, 1.0))
        x = jnp.maximum(
            jnp.dot(feat[...], w1[...], preferred_element_type=jnp.float32)
            + b1[...], 0.0)
        x_ref[...] = x
        g0_ref[...] = x * dinv
        dinv_ref[...] = jnp.broadcast_to(dinv, (B, DEG_W))

    return pl.pallas_call(
        body,
        grid=(nblk,),
        in_specs=[
            pl.BlockSpec((B, DEG_W), lambda i: (i, 0)),
            pl.BlockSpec((B, DEG_W), lambda i, _n=nblk: (i + _n, 0)),
            pl.BlockSpec((B, in_f), lambda i: (i, 0)),
            pl.BlockSpec((in_f, h), lambda i: (0, 0)),
            pl.BlockSpec((1, h), lambda i: (0, 0)),
        ],
        out_specs=[
            pl.BlockSpec((B, h), lambda i: (i, 0)),
            pl.BlockSpec((B, h), lambda i: (i, 0)),
            pl.BlockSpec((B, DEG_W), lambda i: (i, 0)),
        ],
        out_shape=[
            jax.ShapeDtypeStruct((n, h), jnp.float32),
            jax.ShapeDtypeStruct((n, h), jnp.float32),
            jax.ShapeDtypeStruct((n, DEG_W), jnp.float32),
        ],
    )


def _tc_mid(n, d):
    nblk = n // B

    def body(p0, p1, f, dinv16, ad_ref, g_ref):
        dinv = dinv16[:, 0:1]
        ad = (p0[...] + p1[...]) * dinv
        ad_ref[...] = ad
        g_ref[...] = (f[...] - ad) * dinv

    return pl.pallas_call(
        body,
        grid=(nblk,),
        in_specs=[
            pl.BlockSpec((B, d), lambda i: (i, 0)),
            pl.BlockSpec((B, d), lambda i, _n=nblk: (i + _n, 0)),
            pl.BlockSpec((B, d), lambda i: (i, 0)),
            pl.BlockSpec((B, DEG_W), lambda i: (i, 0)),
        ],
        out_specs=[
            pl.BlockSpec((B, d), lambda i: (i, 0)),
            pl.BlockSpec((B, d), lambda i: (i, 0)),
        ],
        out_shape=[
            jax.ShapeDtypeStruct((n, d), jnp.float32),
            jax.ShapeDtypeStruct((n, d), jnp.float32),
        ],
    )


def _tc3(n, h, z, in_f):
    nblk = n // B
    assert in_f == 2 * h

    def body(p0, p1, x, a1d, dinv16, eps, vsum, v12, v2, b2, repw, repb,
             wf, bf, y_ref, gy_ref, kl_ref):
        i = pl.program_id(0)
        dinv = dinv16[:, 0:1]
        a2d = (p0[...] + p1[...]) * dinv
        dot = functools.partial(jnp.dot, preferred_element_type=jnp.float32)
        pos = (dot(x[...], vsum[...]) - dot(a1d[...], v12[...])
               - dot(a2d, v2[...]) + b2[...])
        mu = dot(pos, repw[...]) + repb[...]
        zz = mu + eps[...] * jnp.exp(mu * 0.5)
        y = jnp.maximum(dot(zz, wf[...]) + bf[...], 0.0)
        y_ref[...] = y
        gy_ref[...] = y * dinv

        @pl.when(i == 0)
        def _():
            kl_ref[...] = jnp.zeros((1, 1), jnp.float32)

        kl_ref[...] += jnp.sum(1.0 + mu - mu * mu - jnp.exp(mu)).reshape(1, 1)

    return pl.pallas_call(
        body,
        grid=(nblk,),
        in_specs=[
            pl.BlockSpec((B, h), lambda i: (i, 0)),
            pl.BlockSpec((B, h), lambda i, _n=nblk: (i + _n, 0)),
            pl.BlockSpec((B, h), lambda i: (i, 0)),
            pl.BlockSpec((B, h), lambda i: (i, 0)),
            pl.BlockSpec((B, DEG_W), lambda i: (i, 0)),
            pl.BlockSpec((B, z), lambda i: (i, 0)),
            pl.BlockSpec((h, h), lambda i: (0, 0)),
            pl.BlockSpec((h, h), lambda i: (0, 0)),
            pl.BlockSpec((h, h), lambda i: (0, 0)),
            pl.BlockSpec((1, h), lambda i: (0, 0)),
            pl.BlockSpec((h, z), lambda i: (0, 0)),
            pl.BlockSpec((1, z), lambda i: (0, 0)),
            pl.BlockSpec((z, in_f), lambda i: (0, 0)),
            pl.BlockSpec((1, in_f), lambda i: (0, 0)),
        ],
        out_specs=[
            pl.BlockSpec((B, in_f), lambda i: (i, 0)),
            pl.BlockSpec((B, in_f), lambda i: (i, 0)),
            pl.BlockSpec((1, 1), lambda i: (0, 0)),
        ],
        out_shape=[
            jax.ShapeDtypeStruct((n, in_f), jnp.float32),
            jax.ShapeDtypeStruct((n, in_f), jnp.float32),
            jax.ShapeDtypeStruct((1, 1), jnp.float32),
        ],
    )


def _tc5(n, in_f):
    nblk = n // B

    def body(p0, p1, y, b1d, dinv16, feat, usum, u12, u2, db2, kl, out_ref):
        i = pl.program_id(0)
        dinv = dinv16[:, 0:1]
        b2d = (p0[...] + p1[...]) * dinv
        dot = functools.partial(jnp.dot, preferred_element_type=jnp.float32)
        xr = (dot(y[...], usum[...]) - dot(b1d[...], u12[...])
              - dot(b2d, u2[...]) + db2[...])
        d = xr - feat[...]

        @pl.when(i == 0)
        def _():
            out_ref[...] = -0.5 * kl[...]

        out_ref[...] += jnp.sum(d * d).reshape(1, 1)

    return pl.pallas_call(
        body,
        grid=(nblk,),
        in_specs=[
            pl.BlockSpec((B, in_f), lambda i: (i, 0)),
            pl.BlockSpec((B, in_f), lambda i, _n=nblk: (i + _n, 0)),
            pl.BlockSpec((B, in_f), lambda i: (i, 0)),
            pl.BlockSpec((B, in_f), lambda i: (i, 0)),
            pl.BlockSpec((B, DEG_W), lambda i: (i, 0)),
            pl.BlockSpec((B, in_f), lambda i: (i, 0)),
            pl.BlockSpec((in_f, in_f), lambda i: (0, 0)),
            pl.BlockSpec((in_f, in_f), lambda i: (0, 0)),
            pl.BlockSpec((in_f, in_f), lambda i: (0, 0)),
            pl.BlockSpec((1, in_f), lambda i: (0, 0)),
            pl.BlockSpec((1, 1), lambda i: (0, 0)),
        ],
        out_specs=pl.BlockSpec((1, 1), lambda i: (0, 0)),
        out_shape=jax.ShapeDtypeStruct((1, 1), jnp.float32),
    )


def _fold_theta(w2, d):
    w2r = w2.reshape(3, d, w2.shape[1])

    def mix(coefs):
        return (coefs[0] * w2r[0] + coefs[1] * w2r[1] + coefs[2] * w2r[2])

    vs = [mix([THETA[i][k] for i in range(3)]) for k in range(3)]
    return vs[0] + vs[1] + vs[2], vs[1] + vs[2], vs[2]


def kernel(features, edge_index, enc_W1, enc_b1, enc_W2, enc_b2, rep_W,
           rep_b, rec_W, rec_b, dec_W1, dec_b1, dec_W2, dec_b2, disc_W):
    n, in_f = features.shape
    h = enc_W1.shape[1]
    z = rep_W.shape[1]
    e = edge_index.shape[1]

    n_chunks = -(-e // K)
    n_chunks = ((n_chunks + NW * 8 - 1) // (NW * 8)) * (NW * 8)
    pad = n_chunks * K - e
    src = jnp.concatenate(
        [edge_index[0].astype(jnp.int32), jnp.zeros((pad,), jnp.int32)]
    ).reshape(n_chunks, K)
    dst = jnp.concatenate(
        [edge_index[1].astype(jnp.int32), jnp.full((pad,), n, jnp.int32)]
    ).reshape(n_chunks, K)

    ra = _round8(-(-n // NS))
    zeros_h = jnp.zeros((ra, h), jnp.float32)
    zeros_f = jnp.zeros((ra, in_f), jnp.float32)
    zeros_d = jnp.zeros((ra, DEG_W), jnp.float32)
    ones_d = jnp.ones((K, DEG_W), jnp.float32)

    vsum, v12, v2 = _fold_theta(enc_W2, h)
    usum, u12, u2 = _fold_theta(dec_W2, in_f)
    wf = rec_W @ dec_W1
    bf = (rec_b @ dec_W1 + dec_b1).reshape(1, in_f)
    b1r = enc_b1.reshape(1, h)
    b2r = enc_b2.reshape(1, h)
    db2r = dec_b2.reshape(1, in_f)
    repbr = rep_b.reshape(1, z)

    keps = jax.random.split(jax.random.key(42))[1]
    eps = jax.random.normal(keps, (n, z), dtype=jnp.float32)

    seg_h = _seg_sum_sc(n, n_chunks, h, ring=False)
    seg_f = _seg_sum_sc(n, n_chunks, in_f, ring=False)

    deg_p = _deg_sc(n, n_chunks)(dst, zeros_d, ones_d)
    x, g0, dinv16 = _tc1(n, in_f, h)(deg_p, deg_p, features, enc_W1, b1r)
    a1_p = seg_h(g0, src, dst, zeros_h)
    a1d, g1 = _tc_mid(n, h)(a1_p, a1_p, x, dinv16)
    a2_p = seg_h(g1, src, dst, zeros_h)
    y, gy0, kl = _tc3(n, h, z, in_f)(
        a2_p, a2_p, x, a1d, dinv16, eps, vsum, v12, v2, b2r, rep_W, repbr,
        wf, bf)
    b1_p = seg_f(gy0, src, dst, zeros_f)
    b1d, gy1 = _tc_mid(n, in_f)(b1_p, b1_p, y, dinv16)
    b2_p = seg_f(gy1, src, dst, zeros_f)
    out = _tc5(n, in_f)(b2_p, b2_p, y, b1d, dinv16, features, usum, u12,
                        u2, db2r, kl)
    return out[0, 0]
